# Initial kernel scaffold; baseline (speedup 1.0000x reference)
import jax
import jax.numpy as jnp
from jax.experimental import pallas as pl


def kernel(x, edge_index, label_p, cm, W1, b1, W2, b2):
    raise NotImplementedError("write your pallas kernel here")


# speedup vs baseline: 6.8759x; 6.8759x over previous
import jax
import jax.numpy as jnp
from jax import lax
from jax.experimental import pallas as pl
from jax.experimental.pallas import tpu as pltpu
from jax.experimental.pallas import tpu_sc as plsc

N = 10000
E = 320000
D = 128

NC = 2
NS = 16
NW = NC * NS
EPT = E // NW
CH = 128
NFULL = EPT // CH
REM = EPT - NFULL * CH
L = 16
NP = 10240
RPW = NP // NS


def _make_agg(with_counts: bool):
  mesh = plsc.VectorSubcoreMesh(
      core_axis_name="c", subcore_axis_name="s", num_cores=NC, num_subcores=NS)

  out_type = [jax.ShapeDtypeStruct((NC, NP, D), jnp.float32)]
  scratch = [
      pltpu.VMEM_SHARED((NP, D), jnp.float32),
      pltpu.VMEM((CH,), jnp.int32),
      pltpu.VMEM((CH,), jnp.int32),
      pltpu.VMEM((CH, D), jnp.float32),
      pltpu.VMEM((REM,), jnp.int32),
      pltpu.VMEM((REM,), jnp.int32),
      pltpu.VMEM((REM, D), jnp.float32),
      pltpu.SemaphoreType.DMA,
  ]
  if with_counts:
    out_type.append(jax.ShapeDtypeStruct((NC, NS, NP), jnp.float32))
    scratch += [
        pltpu.VMEM((NP,), jnp.float32),
        pltpu.VMEM((L,), jnp.float32),
    ]

  def body(*refs):
    if with_counts:
      (h, srcs, dsts, zeros, zeros1, ones_h, part, cnt_out,
       acc, src_v, dst_v, rows_v, src_r, dst_r, rows_r, sem,
       cnt_l, ones_v) = refs
    else:
      (h, srcs, dsts, zeros, part,
       acc, src_v, dst_v, rows_v, src_r, dst_r, rows_r, sem) = refs
    c = lax.axis_index("c")
    s = lax.axis_index("s")
    wid = s * NC + c
    rb = s * RPW
    pltpu.sync_copy(zeros.at[pl.ds(rb, RPW), :], acc.at[pl.ds(rb, RPW), :])
    if with_counts:
      pltpu.sync_copy(zeros1, cnt_l)
      pltpu.sync_copy(ones_h, ones_v)
    plsc.subcore_barrier()

    eb = wid * EPT

    def step(j, carry):
      off = pl.multiple_of(eb + j * CH, 8)
      pltpu.sync_copy(srcs.at[pl.ds(off, CH)], src_v)
      pltpu.sync_copy(dsts.at[pl.ds(off, CH)], dst_v)
      pltpu.async_copy(h.at[src_v], rows_v, sem).wait()
      pltpu.sync_copy(rows_v, acc.at[dst_v], add=True)
      if with_counts:
        ones16 = ones_v[...]
        for i in range(CH // L):
          plsc.addupdate_scatter(cnt_l, [dst_v[pl.ds(i * L, L)]], ones16)
      return carry

    lax.fori_loop(0, NFULL, step, 0)

    offr = pl.multiple_of(eb + NFULL * CH, 8)
    pltpu.sync_copy(srcs.at[pl.ds(offr, REM)], src_r)
    pltpu.sync_copy(dsts.at[pl.ds(offr, REM)], dst_r)
    pltpu.async_copy(h.at[src_r], rows_r, sem).wait()
    pltpu.sync_copy(rows_r, acc.at[dst_r], add=True)
    if with_counts:
      plsc.addupdate_scatter(cnt_l, [dst_r[...]], ones_v[...])

    plsc.subcore_barrier()
    pltpu.sync_copy(acc.at[pl.ds(rb, RPW), :], part.at[c, pl.ds(rb, RPW), :])
    if with_counts:
      pltpu.sync_copy(cnt_l, cnt_out.at[c, s])

  return pl.kernel(
      body, out_type=out_type, mesh=mesh, scratch_types=scratch,
      compiler_params=pltpu.CompilerParams(needs_layout_passes=False))


_agg_counts = _make_agg(True)
_agg = _make_agg(False)

BR = 1024


def _lin_body(x_ref, w_ref, b_ref, o_ref):
  o_ref[...] = lax.dot_general(
      x_ref[...], w_ref[...], (((1,), (1,)), ((), ())),
      preferred_element_type=jnp.float32) + b_ref[...]


BRL = 1000

_linear = pl.pallas_call(
    _lin_body,
    grid=(N // BRL,),
    in_specs=[
        pl.BlockSpec((BRL, D), lambda i: (i, 0)),
        pl.BlockSpec((D, D), lambda i: (0, 0)),
        pl.BlockSpec((1, D), lambda i: (0, 0)),
    ],
    out_specs=pl.BlockSpec((BRL, D), lambda i: (i, 0)),
    out_shape=jax.ShapeDtypeStruct((N, D), jnp.float32),
)


def _mid_body(p_ref, c_ref, w_ref, b_ref, o_ref):
  ssum = p_ref[0] + p_ref[1]
  cnt = jnp.sum(c_ref[...], axis=0)[:, None]
  h = jnp.maximum(ssum / jnp.maximum(cnt, 1.0), 0.0)
  o_ref[...] = lax.dot_general(
      h, w_ref[...], (((1,), (1,)), ((), ())),
      preferred_element_type=jnp.float32) + b_ref[...]


_mid = pl.pallas_call(
    _mid_body,
    grid=(NP // BR,),
    in_specs=[
        pl.BlockSpec((NC, BR, D), lambda i: (0, i, 0)),
        pl.BlockSpec((NW, BR), lambda i: (0, i)),
        pl.BlockSpec((D, D), lambda i: (0, 0)),
        pl.BlockSpec((1, D), lambda i: (0, 0)),
    ],
    out_specs=pl.BlockSpec((BR, D), lambda i: (i, 0)),
    out_shape=jax.ShapeDtypeStruct((NP, D), jnp.float32),
)


def _fin_body(p_ref, c_ref, o_ref):
  ssum = p_ref[0] + p_ref[1]
  cnt = jnp.sum(c_ref[...], axis=0)[:, None]
  o_ref[...] = ssum / jnp.maximum(cnt, 1.0)


_fin = pl.pallas_call(
    _fin_body,
    grid=(NP // BR,),
    in_specs=[
        pl.BlockSpec((NC, BR, D), lambda i: (0, i, 0)),
        pl.BlockSpec((NW, BR), lambda i: (0, i)),
    ],
    out_specs=pl.BlockSpec((BR, D), lambda i: (i, 0)),
    out_shape=jax.ShapeDtypeStruct((NP, D), jnp.float32),
)


@jax.jit
def kernel(x, edge_index, label_p, cm, W1, b1, W2, b2):
  src = edge_index[0]
  dst = edge_index[1]
  zeros = jnp.zeros((NP, D), jnp.float32)
  zeros1 = jnp.zeros((NP,), jnp.float32)
  ones = jnp.ones((L,), jnp.float32)
  h1 = _linear(x, W1, b1.reshape(1, D))
  part1, cnt = _agg_counts(h1, src, dst, zeros, zeros1, ones)
  cnt = cnt.reshape(NW, NP)
  h2 = _mid(part1, cnt, W2, b2.reshape(1, D))
  (part2,) = _agg(h2, src, dst, zeros)
  return _fin(part2, cnt)[:N]
